# baseline (speedup 1.0000x reference)
import jax
import jax.numpy as jnp
from jax.experimental import pallas as pl

_L = 2048
_BLK = 256


def _body(con_ref, seqT_ref, seq_ref, out_ref):
    seqT = seqT_ref[...]
    m_r = jnp.max(seqT, axis=1, keepdims=True)
    p_r = jnp.where(
        seqT[:, 0:1] == m_r, 2.0,
        jnp.where(seqT[:, 1:2] == m_r, 3.0,
                  jnp.where(seqT[:, 2:3] == m_r, 5.0, 7.0)))
    seq = seq_ref[...]
    m_c = jnp.max(seq, axis=0, keepdims=True)
    p_c = jnp.where(
        seq[0:1, :] == m_c, 2.0,
        jnp.where(seq[1:2, :] == m_c, 3.0,
                  jnp.where(seq[2:3, :] == m_c, 5.0, 7.0)))
    prod = p_r * p_c
    keep = (prod == 14.0) | (prod == 15.0) | (prod == 35.0)
    keep = keep | (m_r < 1.0) | (m_c < 1.0)
    out_ref[...] = jnp.where(keep, con_ref[...], 0.0)


def kernel(con, feat):
    con2 = con.reshape(_L, _L)
    seq = feat[0, :4, :, 0]
    seqT = jnp.transpose(seq)
    out = pl.pallas_call(
        _body,
        grid=(_L // _BLK,),
        in_specs=[
            pl.BlockSpec((_BLK, _L), lambda i: (i, 0)),
            pl.BlockSpec((_BLK, 4), lambda i: (i, 0)),
            pl.BlockSpec((4, _L), lambda i: (0, 0)),
        ],
        out_specs=pl.BlockSpec((_BLK, _L), lambda i: (i, 0)),
        out_shape=jax.ShapeDtypeStruct((_L, _L), jnp.float32),
    )(con2, seqT, seq)
    return out.reshape(con.shape)


# speedup vs baseline: 1.0814x; 1.0814x over previous
import jax
import jax.numpy as jnp
from jax.experimental import pallas as pl

_L = 2048
_BLK = 256


def _col_factors(seq):
    m = jnp.max(seq, axis=0, keepdims=True)
    a = seq[0:1, :] == m
    c = (seq[1:2, :] == m) & ~a
    g = (seq[2:3, :] == m) & ~a & ~c
    u = ~(a | c | g)
    nd = jnp.where(m < 1.0, 0.0, 1.0)
    ca = jnp.where(a, nd, 0.0)
    cc = jnp.where(c, nd, 0.0)
    cg = jnp.where(g, nd, 0.0)
    cu = jnp.where(u, nd, 0.0)
    zero = jnp.zeros_like(nd)
    return jnp.concatenate(
        [nd, -cu, -cg, -(cc + cu), -(ca + cg), zero, zero, zero], axis=0)


def _row_factors(seqT):
    m = jnp.max(seqT, axis=1, keepdims=True)
    a = seqT[:, 0:1] == m
    c = (seqT[:, 1:2] == m) & ~a
    g = (seqT[:, 2:3] == m) & ~a & ~c
    u = ~(a | c | g)
    nd = jnp.where(m < 1.0, 0.0, 1.0)
    ra = jnp.where(a, nd, 0.0)
    rc = jnp.where(c, nd, 0.0)
    rg = jnp.where(g, nd, 0.0)
    ru = jnp.where(u, nd, 0.0)
    zero = jnp.zeros_like(nd)
    return jnp.concatenate([nd, ra, rc, rg, ru, zero, zero, zero], axis=1)


def _body(con_ref, seqT_ref, seq_ref, out_ref):
    u = _row_factors(seqT_ref[...])
    v = _col_factors(seq_ref[...])
    notkeep = jax.lax.dot_general(
        u, v, (((1,), (0,)), ((), ())),
        preferred_element_type=jnp.float32)
    con = con_ref[...]
    out_ref[...] = con - con * notkeep


def kernel(con, feat):
    con2 = con.reshape(_L, _L)
    seq = feat[0, :4, :, 0]
    seqT = jnp.transpose(seq)
    out = pl.pallas_call(
        _body,
        grid=(_L // _BLK,),
        in_specs=[
            pl.BlockSpec((_BLK, _L), lambda i: (i, 0)),
            pl.BlockSpec((_BLK, 4), lambda i: (i, 0)),
            pl.BlockSpec((4, _L), lambda i: (0, 0)),
        ],
        out_specs=pl.BlockSpec((_BLK, _L), lambda i: (i, 0)),
        out_shape=jax.ShapeDtypeStruct((_L, _L), jnp.float32),
    )(con2, seqT, seq)
    return out.reshape(con.shape)


# speedup vs baseline: 1.2254x; 1.1332x over previous
import jax
import jax.numpy as jnp
from jax.experimental import pallas as pl

_L = 2048
_BLK = 1024


def _col_factors(seq):
    m = jnp.max(seq, axis=0, keepdims=True)
    a = seq[0:1, :] == m
    c = (seq[1:2, :] == m) & ~a
    g = (seq[2:3, :] == m) & ~a & ~c
    u = ~(a | c | g)
    nd = jnp.where(m < 1.0, 0.0, 1.0)
    ca = jnp.where(a, nd, 0.0)
    cc = jnp.where(c, nd, 0.0)
    cg = jnp.where(g, nd, 0.0)
    cu = jnp.where(u, nd, 0.0)
    one = jnp.ones_like(nd)
    zero = jnp.zeros_like(nd)
    return jnp.concatenate(
        [one, nd, cu, cg, cc + cu, ca + cg, zero, zero], axis=0)


def _row_factors(seqT):
    m = jnp.max(seqT, axis=1, keepdims=True)
    a = seqT[:, 0:1] == m
    c = (seqT[:, 1:2] == m) & ~a
    g = (seqT[:, 2:3] == m) & ~a & ~c
    u = ~(a | c | g)
    nd = jnp.where(m < 1.0, 0.0, 1.0)
    ra = jnp.where(a, nd, 0.0)
    rc = jnp.where(c, nd, 0.0)
    rg = jnp.where(g, nd, 0.0)
    ru = jnp.where(u, nd, 0.0)
    one = jnp.ones_like(nd)
    zero = jnp.zeros_like(nd)
    return jnp.concatenate([one, -nd, ra, rc, rg, ru, zero, zero], axis=1)


def _body(con_ref, seqT_ref, seq_ref, out_ref):
    u = _row_factors(seqT_ref[...])
    v = _col_factors(seq_ref[...])
    keep = jax.lax.dot_general(
        u, v, (((1,), (0,)), ((), ())),
        preferred_element_type=jnp.float32)
    out_ref[...] = con_ref[...] * keep


def kernel(con, feat):
    con2 = con.reshape(_L, _L)
    seq = feat[0, :4, :, 0]
    seqT = jnp.transpose(seq)
    out = pl.pallas_call(
        _body,
        grid=(_L // _BLK,),
        in_specs=[
            pl.BlockSpec((_BLK, _L), lambda i: (i, 0)),
            pl.BlockSpec((_BLK, 4), lambda i: (i, 0)),
            pl.BlockSpec((4, _L), lambda i: (0, 0)),
        ],
        out_specs=pl.BlockSpec((_BLK, _L), lambda i: (i, 0)),
        out_shape=jax.ShapeDtypeStruct((_L, _L), jnp.float32),
    )(con2, seqT, seq)
    return out.reshape(con.shape)


# speedup vs baseline: 1.4572x; 1.1892x over previous
import jax
import jax.numpy as jnp
from jax.experimental import pallas as pl

_L = 2048
_BLK = 1024


def _body(con_ref, seqT_ref, seq_ref, out_ref):
    seqT = seqT_ref[...]
    m_r = jnp.max(seqT, axis=1, keepdims=True)
    a_r = seqT[:, 0:1] == m_r
    c_r = (seqT[:, 1:2] == m_r) & ~a_r
    g_r = (seqT[:, 2:3] == m_r) & ~a_r & ~c_r
    allowed = jnp.where(a_r, 8, jnp.where(c_r, 4, jnp.where(g_r, 10, 5)))
    rowmask = jnp.where(m_r < 1.0, 15, allowed) + 16

    seq = seq_ref[...]
    m_c = jnp.max(seq, axis=0, keepdims=True)
    a_c = seq[0:1, :] == m_c
    c_c = (seq[1:2, :] == m_c) & ~a_c
    g_c = (seq[2:3, :] == m_c) & ~a_c & ~c_c
    colbit = jnp.where(
        m_c < 1.0, 16,
        jnp.where(a_c, 1, jnp.where(c_c, 2, jnp.where(g_c, 4, 8))))

    keep = (rowmask & colbit) != 0
    out_ref[...] = jnp.where(keep, con_ref[...], 0.0)


def kernel(con, feat):
    con2 = con.reshape(_L, _L)
    seq = feat[0, :4, :, 0]
    seqT = jnp.transpose(seq)
    out = pl.pallas_call(
        _body,
        grid=(_L // _BLK,),
        in_specs=[
            pl.BlockSpec((_BLK, _L), lambda i: (i, 0)),
            pl.BlockSpec((_BLK, 4), lambda i: (i, 0)),
            pl.BlockSpec((4, _L), lambda i: (0, 0)),
        ],
        out_specs=pl.BlockSpec((_BLK, _L), lambda i: (i, 0)),
        out_shape=jax.ShapeDtypeStruct((_L, _L), jnp.float32),
    )(con2, seqT, seq)
    return out.reshape(con.shape)
